# baseline (speedup 1.0000x reference)
import jax
import jax.numpy as jnp
from jax.experimental import pallas as pl
from jax.experimental.pallas import tpu as pltpu


def _dma_body(x_ref, lo_ref, mid_ref, hi_ref, s0, s1, s2):
    c0 = pltpu.make_async_copy(x_ref.at[:, 0:48, :], lo_ref, s0)
    c1 = pltpu.make_async_copy(x_ref.at[:, 48:96, :], mid_ref, s1)
    c2 = pltpu.make_async_copy(x_ref.at[:, 96:128, :], hi_ref, s2)
    c0.start()
    c1.start()
    c2.start()
    c0.wait()
    c1.wait()
    c2.wait()


def kernel(x, idx_low, idx_mid, idx_high):
    B, _, R, C = x.shape
    x3 = x.reshape(B, R, C)
    lo, mid, hi = pl.pallas_call(
        _dma_body,
        in_specs=[pl.BlockSpec(memory_space=pl.ANY)],
        out_specs=(
            pl.BlockSpec(memory_space=pl.ANY),
            pl.BlockSpec(memory_space=pl.ANY),
            pl.BlockSpec(memory_space=pl.ANY),
        ),
        out_shape=(
            jax.ShapeDtypeStruct((B, 48, C), x.dtype),
            jax.ShapeDtypeStruct((B, 48, C), x.dtype),
            jax.ShapeDtypeStruct((B, 32, C), x.dtype),
        ),
        scratch_shapes=[pltpu.SemaphoreType.DMA] * 3,
    )(x3)
    return lo.reshape(B, 1, 48, C), mid, hi.reshape(B, 1, 32, C)


# speedup vs baseline: 35.3638x; 35.3638x over previous
import functools

import jax
import jax.numpy as jnp
from jax import lax
from jax.experimental import pallas as pl
from jax.experimental.pallas import tpu as pltpu
from jax.experimental.pallas import tpu_sc as plsc

_CHUNK = 8
_BANDS = ((0, 0, 48), (1, 48, 48), (2, 96, 32))


def _sc_body(x_hbm, lo_hbm, mid_hbm, hi_hbm, buf0, buf1, si0, si1, so0, so1):
    c = lax.axis_index("c")
    s = lax.axis_index("s")
    w = s * 2 + c
    bufs = (buf0, buf1)
    in_sems = (si0, si1)
    out_sems = (so0, so1)
    outs = (lo_hbm, mid_hbm, hi_hbm)

    tasks = []
    for lb in range(2):
        for oid, src0, nrows in _BANDS:
            for k in range(nrows // _CHUNK):
                tasks.append((lb, oid, src0 + k * _CHUNK, k * _CHUNK))

    def make_in(i):
        lb, _, r0, _ = tasks[i]
        return pltpu.make_async_copy(
            x_hbm.at[w * 2 + lb, pl.ds(r0, _CHUNK)], bufs[i % 2], in_sems[i % 2]
        )

    def make_out(i):
        lb, oid, _, d0 = tasks[i]
        return pltpu.make_async_copy(
            bufs[i % 2], outs[oid].at[w * 2 + lb, pl.ds(d0, _CHUNK)], out_sems[i % 2]
        )

    n = len(tasks)
    make_in(0).start()
    make_in(1).start()
    for i in range(n):
        make_in(i).wait()
        make_out(i).start()
        make_out(i).wait()
        if i + 2 < n:
            make_in(i + 2).start()


def kernel(x, idx_low, idx_mid, idx_high):
    B, _, R, C = x.shape
    x3 = x.reshape(B, R, C)
    mesh = plsc.VectorSubcoreMesh(core_axis_name="c", subcore_axis_name="s")
    run = functools.partial(
        pl.kernel,
        mesh=mesh,
        out_type=(
            jax.ShapeDtypeStruct((B, 48, C), x.dtype),
            jax.ShapeDtypeStruct((B, 48, C), x.dtype),
            jax.ShapeDtypeStruct((B, 32, C), x.dtype),
        ),
        scratch_types=[
            pltpu.VMEM((_CHUNK, C), x.dtype),
            pltpu.VMEM((_CHUNK, C), x.dtype),
            pltpu.SemaphoreType.DMA,
            pltpu.SemaphoreType.DMA,
            pltpu.SemaphoreType.DMA,
            pltpu.SemaphoreType.DMA,
        ],
    )(_sc_body)
    lo, mid, hi = run(x3)
    return lo.reshape(B, 1, 48, C), mid, hi.reshape(B, 1, 32, C)


# speedup vs baseline: 35.9064x; 1.0153x over previous
import functools

import jax
import jax.numpy as jnp
from jax import lax
from jax.experimental import pallas as pl
from jax.experimental.pallas import tpu as pltpu
from jax.experimental.pallas import tpu_sc as plsc

_CHUNK = 8
_MID0, _MIDN = 48, 48


def _sc_mid_body(x_hbm, mid_hbm, buf0, buf1, si0, si1, so0, so1):
    c = lax.axis_index("c")
    s = lax.axis_index("s")
    w = s * 2 + c
    bufs = (buf0, buf1)
    in_sems = (si0, si1)
    out_sems = (so0, so1)

    tasks = []
    for lb in range(2):
        for k in range(_MIDN // _CHUNK):
            tasks.append((lb, k * _CHUNK))

    def make_in(i):
        lb, d0 = tasks[i]
        return pltpu.make_async_copy(
            x_hbm.at[w * 2 + lb, pl.ds(_MID0 + d0, _CHUNK)],
            bufs[i % 2],
            in_sems[i % 2],
        )

    def make_out(i):
        lb, d0 = tasks[i]
        return pltpu.make_async_copy(
            bufs[i % 2], mid_hbm.at[w * 2 + lb, pl.ds(d0, _CHUNK)], out_sems[i % 2]
        )

    n = len(tasks)
    make_in(0).start()
    make_in(1).start()
    for i in range(n):
        make_in(i).wait()
        make_out(i).start()
        make_out(i).wait()
        if i + 2 < n:
            make_in(i + 2).start()


def _tc_lohi_body(xlo_ref, xhi_ref, lo_ref, hi_ref):
    lo_ref[...] = xlo_ref[...]
    hi_ref[...] = xhi_ref[...]


def kernel(x, idx_low, idx_mid, idx_high):
    B, _, R, C = x.shape
    x3 = x.reshape(B, R, C)

    mesh = plsc.VectorSubcoreMesh(core_axis_name="c", subcore_axis_name="s")
    sc_mid = functools.partial(
        pl.kernel,
        mesh=mesh,
        out_type=jax.ShapeDtypeStruct((B, _MIDN, C), x.dtype),
        scratch_types=[
            pltpu.VMEM((_CHUNK, C), x.dtype),
            pltpu.VMEM((_CHUNK, C), x.dtype),
            pltpu.SemaphoreType.DMA,
            pltpu.SemaphoreType.DMA,
            pltpu.SemaphoreType.DMA,
            pltpu.SemaphoreType.DMA,
        ],
    )(_sc_mid_body)
    mid = sc_mid(x3)

    lo, hi = pl.pallas_call(
        _tc_lohi_body,
        grid=(B,),
        in_specs=[
            pl.BlockSpec((1, 48, C), lambda b: (b, 0, 0)),
            pl.BlockSpec((1, 32, C), lambda b: (b, 3, 0)),
        ],
        out_specs=(
            pl.BlockSpec((1, 48, C), lambda b: (b, 0, 0)),
            pl.BlockSpec((1, 32, C), lambda b: (b, 0, 0)),
        ),
        out_shape=(
            jax.ShapeDtypeStruct((B, 48, C), x.dtype),
            jax.ShapeDtypeStruct((B, 32, C), x.dtype),
        ),
    )(x3, x3)

    return lo.reshape(B, 1, 48, C), mid, hi.reshape(B, 1, 32, C)


# speedup vs baseline: 36.1903x; 1.0079x over previous
import functools

import jax
import jax.numpy as jnp
from jax import lax
from jax.experimental import pallas as pl
from jax.experimental.pallas import tpu as pltpu
from jax.experimental.pallas import tpu_sc as plsc

_CHUNK = 8
_HI0, _HIN = 96, 32


def _sc_hi_body(x_hbm, hi_hbm, buf0, buf1, si0, si1, so0, so1):
    c = lax.axis_index("c")
    s = lax.axis_index("s")
    w = s * 2 + c
    bufs = (buf0, buf1)
    in_sems = (si0, si1)
    out_sems = (so0, so1)

    tasks = []
    for lb in range(2):
        for k in range(_HIN // _CHUNK):
            tasks.append((lb, k * _CHUNK))

    def make_in(i):
        lb, d0 = tasks[i]
        return pltpu.make_async_copy(
            x_hbm.at[w * 2 + lb, pl.ds(_HI0 + d0, _CHUNK)], bufs[i % 2], in_sems[i % 2]
        )

    def make_out(i):
        lb, d0 = tasks[i]
        return pltpu.make_async_copy(
            bufs[i % 2], hi_hbm.at[w * 2 + lb, pl.ds(d0, _CHUNK)], out_sems[i % 2]
        )

    n = len(tasks)
    make_in(0).start()
    make_in(1).start()
    for i in range(n):
        make_in(i).wait()
        make_out(i).start()
        make_out(i).wait()
        if i + 2 < n:
            make_in(i + 2).start()


def _tc_lomid_body(x_ref, lo_ref, mid_ref):
    lo_ref[...] = x_ref[:, 0:48, :]
    mid_ref[...] = x_ref[:, 48:96, :]


def kernel(x, idx_low, idx_mid, idx_high):
    B, _, R, C = x.shape
    x3 = x.reshape(B, R, C)

    mesh = plsc.VectorSubcoreMesh(core_axis_name="c", subcore_axis_name="s")
    sc_hi = functools.partial(
        pl.kernel,
        mesh=mesh,
        out_type=jax.ShapeDtypeStruct((B, _HIN, C), x.dtype),
        scratch_types=[
            pltpu.VMEM((_CHUNK, C), x.dtype),
            pltpu.VMEM((_CHUNK, C), x.dtype),
            pltpu.SemaphoreType.DMA,
            pltpu.SemaphoreType.DMA,
            pltpu.SemaphoreType.DMA,
            pltpu.SemaphoreType.DMA,
        ],
    )(_sc_hi_body)
    hi = sc_hi(x3)

    lo, mid = pl.pallas_call(
        _tc_lomid_body,
        grid=(B,),
        in_specs=[pl.BlockSpec((1, 96, C), lambda b: (b, 0, 0))],
        out_specs=(
            pl.BlockSpec((1, 48, C), lambda b: (b, 0, 0)),
            pl.BlockSpec((1, 48, C), lambda b: (b, 0, 0)),
        ),
        out_shape=(
            jax.ShapeDtypeStruct((B, 48, C), x.dtype),
            jax.ShapeDtypeStruct((B, 48, C), x.dtype),
        ),
    )(x3)

    return lo.reshape(B, 1, 48, C), mid, hi.reshape(B, 1, 32, C)


# speedup vs baseline: 42.4504x; 1.1730x over previous
import jax
import jax.numpy as jnp
from jax.experimental import pallas as pl
from jax.experimental.pallas import tpu as pltpu

_NBUF = 4
_BANDS = ((0, 0, 48), (1, 48, 48), (2, 96, 32))


def _ring_body(x_ref, lo_ref, mid_ref, hi_ref, *rest):
    bufs = rest[:_NBUF]
    in_sems = rest[_NBUF : 2 * _NBUF]
    out_sems = rest[2 * _NBUF :]
    outs = (lo_ref, mid_ref, hi_ref)
    B = x_ref.shape[0]

    def make_in(b):
        j = b % _NBUF
        return pltpu.make_async_copy(x_ref.at[b], bufs[j], in_sems[j])

    def make_out(b, band):
        j = b % _NBUF
        oid, r0, nr = band
        return pltpu.make_async_copy(
            bufs[j].at[pl.ds(r0, nr)], outs[oid].at[b], out_sems[j]
        )

    for b in range(min(_NBUF, B)):
        make_in(b).start()
    for b in range(B):
        make_in(b).wait()
        for band in _BANDS:
            make_out(b, band).start()
        if b + _NBUF < B:
            for band in _BANDS:
                make_out(b, band).wait()
            make_in(b + _NBUF).start()
        else:
            for band in _BANDS:
                make_out(b, band).wait()


def kernel(x, idx_low, idx_mid, idx_high):
    B, _, R, C = x.shape
    x3 = x.reshape(B, R, C)
    lo, mid, hi = pl.pallas_call(
        _ring_body,
        in_specs=[pl.BlockSpec(memory_space=pl.ANY)],
        out_specs=(
            pl.BlockSpec(memory_space=pl.ANY),
            pl.BlockSpec(memory_space=pl.ANY),
            pl.BlockSpec(memory_space=pl.ANY),
        ),
        out_shape=(
            jax.ShapeDtypeStruct((B, 48, C), x.dtype),
            jax.ShapeDtypeStruct((B, 48, C), x.dtype),
            jax.ShapeDtypeStruct((B, 32, C), x.dtype),
        ),
        scratch_shapes=(
            [pltpu.VMEM((R, C), x.dtype) for _ in range(_NBUF)]
            + [pltpu.SemaphoreType.DMA] * (2 * _NBUF)
        ),
    )(x3)
    return lo.reshape(B, 1, 48, C), mid, hi.reshape(B, 1, 32, C)
